# initial kernel scaffold (unmeasured)
import jax
import jax.numpy as jnp
from jax import lax
from jax.experimental import pallas as pl
from jax.experimental.pallas import tpu as pltpu


def kernel(
    x,
):
    def body(*refs):
        pass

    out_shape = jax.ShapeDtypeStruct(..., jnp.float32)
    return pl.pallas_call(body, out_shape=out_shape)(...)


# baseline (device time: 20699 ns/iter reference)
import jax
import jax.numpy as jnp
from jax import lax
from jax.experimental import pallas as pl
from jax.experimental.pallas import tpu as pltpu

N_DEV = 4


def kernel(x):
    m, n_per = x.shape

    def body(x_ref, out_ref, local_stats, stats_ref, send_sems, recv_sems):
        my = lax.axis_index("i")

        xv = x_ref[:, :]
        lmax = jnp.max(xv, axis=1, keepdims=True)
        e = jnp.exp(xv - lmax)
        lsum = jnp.sum(e, axis=1, keepdims=True)
        local_stats[:, 0:1] = lmax
        local_stats[:, 1:2] = lsum

        barrier_sem = pltpu.get_barrier_semaphore()
        for k in range(1, N_DEV):
            peer = (my + k) % N_DEV
            pl.semaphore_signal(
                barrier_sem, inc=1,
                device_id=(peer,), device_id_type=pl.DeviceIdType.MESH,
            )
        pl.semaphore_wait(barrier_sem, N_DEV - 1)

        rdmas = []
        for k in range(1, N_DEV):
            peer = (my + k) % N_DEV
            slot = N_DEV - k
            rdma = pltpu.make_async_remote_copy(
                src_ref=local_stats,
                dst_ref=stats_ref.at[slot],
                send_sem=send_sems.at[k - 1],
                recv_sem=recv_sems.at[slot],
                device_id=(peer,),
                device_id_type=pl.DeviceIdType.MESH,
            )
            rdma.start()
            rdmas.append(rdma)

        for rdma in rdmas:
            rdma.wait()

        gm = lmax
        gs = lsum
        for slot in range(1, N_DEV):
            rm = stats_ref[slot, :, 0:1]
            rs = stats_ref[slot, :, 1:2]
            nm = jnp.maximum(gm, rm)
            gs = gs * jnp.exp(gm - nm) + rs * jnp.exp(rm - nm)
            gm = nm

        out_ref[:, :] = e * (jnp.exp(lmax - gm) / gs)

    return pl.pallas_call(
        body,
        out_shape=jax.ShapeDtypeStruct((m, n_per), x.dtype),
        in_specs=[pl.BlockSpec(memory_space=pltpu.VMEM)],
        out_specs=pl.BlockSpec(memory_space=pltpu.VMEM),
        scratch_shapes=[
            pltpu.VMEM((m, 2), jnp.float32),
            pltpu.VMEM((N_DEV, m, 2), jnp.float32),
            pltpu.SemaphoreType.DMA((N_DEV - 1,)),
            pltpu.SemaphoreType.DMA((N_DEV,)),
        ],
        compiler_params=pltpu.CompilerParams(collective_id=0),
    )(x)


# device time: 9467 ns/iter; 2.1864x vs baseline; 2.1864x over previous
import jax
import jax.numpy as jnp
from jax import lax
from jax.experimental import pallas as pl
from jax.experimental.pallas import tpu as pltpu

N_DEV = 4


def kernel(x):
    m, n_per = x.shape

    def body(x_ref, out_ref, local_stats, stats_ref, send_sems, recv_sems):
        my = lax.axis_index("i")

        xv = x_ref[:, :]
        lmax = jnp.max(xv, axis=1, keepdims=True)
        e = jnp.exp(xv - lmax)
        lsum = jnp.sum(e, axis=1, keepdims=True)
        local_stats[0:1, :] = lmax.reshape(1, m)
        local_stats[1:2, :] = lsum.reshape(1, m)

        barrier_sem = pltpu.get_barrier_semaphore()
        for k in range(1, N_DEV):
            peer = (my + k) % N_DEV
            pl.semaphore_signal(
                barrier_sem, inc=1,
                device_id=(peer,), device_id_type=pl.DeviceIdType.MESH,
            )
        pl.semaphore_wait(barrier_sem, N_DEV - 1)

        rdmas = []
        for k in range(1, N_DEV):
            peer = (my + k) % N_DEV
            slot = N_DEV - k
            rdma = pltpu.make_async_remote_copy(
                src_ref=local_stats,
                dst_ref=stats_ref.at[slot],
                send_sem=send_sems.at[k - 1],
                recv_sem=recv_sems.at[slot],
                device_id=(peer,),
                device_id_type=pl.DeviceIdType.MESH,
            )
            rdma.start()
            rdmas.append(rdma)

        for rdma in rdmas:
            rdma.wait()

        lmax_row = local_stats[0:1, :]
        gm = lmax_row
        gs = local_stats[1:2, :]
        for slot in range(1, N_DEV):
            rm = stats_ref[slot, 0:1, :]
            rs = stats_ref[slot, 1:2, :]
            nm = jnp.maximum(gm, rm)
            gs = gs * jnp.exp(gm - nm) + rs * jnp.exp(rm - nm)
            gm = nm

        scale = jnp.exp(lmax_row - gm) / gs
        out_ref[:, :] = e * scale.reshape(m, 1)

    return pl.pallas_call(
        body,
        out_shape=jax.ShapeDtypeStruct((m, n_per), x.dtype),
        in_specs=[pl.BlockSpec(memory_space=pltpu.VMEM)],
        out_specs=pl.BlockSpec(memory_space=pltpu.VMEM),
        scratch_shapes=[
            pltpu.VMEM((2, m), jnp.float32),
            pltpu.VMEM((N_DEV, 2, m), jnp.float32),
            pltpu.SemaphoreType.DMA((N_DEV - 1,)),
            pltpu.SemaphoreType.DMA((N_DEV,)),
        ],
        compiler_params=pltpu.CompilerParams(collective_id=0),
    )(x)


# device time: 9138 ns/iter; 2.2652x vs baseline; 1.0360x over previous
import jax
import jax.numpy as jnp
from jax import lax
from jax.experimental import pallas as pl
from jax.experimental.pallas import tpu as pltpu

N_DEV = 4
N_HALF = 2


def kernel(x):
    m, n_per = x.shape
    mh = m // N_HALF

    def body(x_ref, out_ref, local_stats, stats_ref, send_sems, recv_sems):
        my = lax.axis_index("i")

        barrier_sem = pltpu.get_barrier_semaphore()
        for k in range(1, N_DEV):
            peer = (my + k) % N_DEV
            pl.semaphore_signal(
                barrier_sem, inc=1,
                device_id=(peer,), device_id_type=pl.DeviceIdType.MESH,
            )

        def local_pass(h):
            xv = x_ref[pl.ds(h * mh, mh), :]
            lmax = jnp.max(xv, axis=1, keepdims=True)
            e = jnp.exp(xv - lmax)
            lsum = jnp.sum(e, axis=1, keepdims=True)
            local_stats[h, 0:1, :] = lmax.reshape(1, mh)
            local_stats[h, 1:2, :] = lsum.reshape(1, mh)
            return e

        def send_stats(h):
            rdmas = []
            for k in range(1, N_DEV):
                peer = (my + k) % N_DEV
                slot = N_DEV - k
                rdma = pltpu.make_async_remote_copy(
                    src_ref=local_stats.at[h],
                    dst_ref=stats_ref.at[slot, h],
                    send_sem=send_sems.at[h * (N_DEV - 1) + k - 1],
                    recv_sem=recv_sems.at[slot, h],
                    device_id=(peer,),
                    device_id_type=pl.DeviceIdType.MESH,
                )
                rdma.start()
                rdmas.append(rdma)
            return rdmas

        def combine_and_scale(h, e, rdmas):
            for rdma in rdmas:
                rdma.wait()
            lmax_row = local_stats[h, 0:1, :]
            gm = lmax_row
            gs = local_stats[h, 1:2, :]
            for slot in range(1, N_DEV):
                rm = stats_ref[slot, h, 0:1, :]
                rs = stats_ref[slot, h, 1:2, :]
                nm = jnp.maximum(gm, rm)
                gs = gs * jnp.exp(gm - nm) + rs * jnp.exp(rm - nm)
                gm = nm
            scale = jnp.exp(lmax_row - gm) / gs
            out_ref[pl.ds(h * mh, mh), :] = e * scale.reshape(mh, 1)

        e0 = local_pass(0)
        pl.semaphore_wait(barrier_sem, N_DEV - 1)
        rdmas0 = send_stats(0)
        e1 = local_pass(1)
        rdmas1 = send_stats(1)
        combine_and_scale(0, e0, rdmas0)
        combine_and_scale(1, e1, rdmas1)

    return pl.pallas_call(
        body,
        out_shape=jax.ShapeDtypeStruct((m, n_per), x.dtype),
        in_specs=[pl.BlockSpec(memory_space=pltpu.VMEM)],
        out_specs=pl.BlockSpec(memory_space=pltpu.VMEM),
        scratch_shapes=[
            pltpu.VMEM((N_HALF, 2, mh), jnp.float32),
            pltpu.VMEM((N_DEV, N_HALF, 2, mh), jnp.float32),
            pltpu.SemaphoreType.DMA((N_HALF * (N_DEV - 1),)),
            pltpu.SemaphoreType.DMA((N_DEV, N_HALF)),
        ],
        compiler_params=pltpu.CompilerParams(collective_id=0),
    )(x)
